# baseline (speedup 1.0000x reference)
import functools

import jax
import jax.numpy as jnp
from jax import lax
from jax.experimental import pallas as pl
from jax.experimental.pallas import tpu as pltpu
from jax.experimental.pallas import tpu_sc as plsc

ALPHA = 0.2
NC = 2
NS = 16
NW = NC * NS
CH = 40
NPHASE = 2
F = 128


def _tc_mm_body(h_ref, w_ref, a_ref, wh_ref, s_ref):
    hh = h_ref[...]
    ww = w_ref[...]
    wh = lax.dot_general(hh, ww, (((1,), (1,)), ((), ())),
                         preferred_element_type=jnp.float32)
    wh_ref[...] = wh
    am = a_ref[...].reshape(2, F)
    s_ref[...] = lax.dot_general(am, wh, (((1,), (1,)), ((), ())),
                                 preferred_element_type=jnp.float32)


def _tc_add_body(p_ref, o_ref):
    o_ref[...] = p_ref[0] + p_ref[1]


def _sc_scores_body(row_hbm, col_hbm, s_hbm, e_hbm, part_hbm,
                    row_v, col_v, s1_v, s2_v, e_v, part_v):
    ept = row_v.shape[0]
    cid = lax.axis_index("c")
    sid = lax.axis_index("s")
    w = cid * NS + sid

    pltpu.sync_copy(row_hbm.at[w], row_v)
    pltpu.sync_copy(col_hbm.at[w], col_v)
    pltpu.sync_copy(s_hbm.at[0], s1_v)
    pltpu.sync_copy(s_hbm.at[1], s2_v)

    neg = jnp.full((16,), -1e30, jnp.float32)

    def pass1(i, m_acc):
        r = row_v[pl.ds(i * 16, 16)]
        c = col_v[pl.ds(i * 16, 16)]
        sr = plsc.load_gather(s1_v, [r])
        sc = plsc.load_gather(s2_v, [c])
        x = sr + sc
        e = jnp.where(x > 0, x, ALPHA * x)
        e_v[pl.ds(i * 16, 16)] = e
        return jnp.maximum(m_acc, e)

    m_vec = lax.fori_loop(0, ept // 16, pass1, neg)
    m16 = jnp.full((16,), jnp.max(m_vec))

    def pass2(i, s_acc):
        return s_acc + jnp.exp(e_v[pl.ds(i * 16, 16)] - m16)

    s_vec = lax.fori_loop(0, ept // 16, pass2, jnp.zeros((16,), jnp.float32))
    part_v[0, :] = m16
    part_v[1, :] = jnp.full((16,), jnp.sum(s_vec))

    pltpu.sync_copy(e_v, e_hbm.at[w])
    pltpu.sync_copy(part_v, part_hbm.at[w])


def _sc_aggr_body(wh_hbm, e_hbm, part_hbm, row3_hbm, col3_hbm, pout_hbm,
                  acc, e_v, row_v, col_v, g0, g1, s0, s1_, part_v,
                  gs0, gs1, ss0, ss1):
    n = acc.shape[0]
    ept = e_v.shape[0]
    nph = row3_hbm.shape[1]
    nphase = row3_hbm.shape[0] // NW
    rows_per_tile = n // NS
    cid = lax.axis_index("c")
    sid = lax.axis_index("s")
    w = cid * NS + sid
    gbuf = (g0, g1)
    sbuf = (s0, s1_)
    gs = (gs0, gs1)
    ss = (ss0, ss1)

    pltpu.sync_copy(e_hbm.at[w], e_v)
    pltpu.sync_copy(part_hbm, part_v)

    def mred(t, m_acc):
        return jnp.maximum(m_acc, part_v[t, 0, :])

    m_all = lax.fori_loop(0, NW, mred, jnp.full((16,), -1e30, jnp.float32))
    m16 = jnp.full((16,), jnp.max(m_all))

    def sred(t, s_acc):
        return s_acc + part_v[t, 1, :] * jnp.exp(part_v[t, 0, :] - m16)

    s_all = lax.fori_loop(0, NW, sred, jnp.zeros((16,), jnp.float32))
    inv_s = 1.0 / s_all

    def att_fill(i, _):
        v = e_v[pl.ds(i * 16, 16)]
        e_v[pl.ds(i * 16, 16)] = jnp.exp(v - m16) * inv_s
        return 0

    lax.fori_loop(0, ept // 16, att_fill, 0)

    def zfill(i, _):
        for q in range(F // 16):
            s0[i, pl.ds(q * 16, 16)] = jnp.zeros((16,), jnp.float32)
        return 0

    lax.fori_loop(0, CH, zfill, 0)

    def zcopy(k, _):
        pltpu.sync_copy(s0, acc.at[pl.ds(sid * rows_per_tile + k * CH, CH)])
        return 0

    lax.fori_loop(0, rows_per_tile // CH, zcopy, 0)
    rem = rows_per_tile % CH
    if rem:
        pltpu.sync_copy(
            s0.at[pl.ds(0, rem)],
            acc.at[pl.ds(sid * rows_per_tile + CH * (rows_per_tile // CH),
                         rem)])
    plsc.subcore_barrier()

    def scale_chunk(b, ch, base):
        unroll = 4

        def scale(k4, _):
            for u in range(unroll):
                k = k4 * unroll + u
                asp = plsc.load_gather(
                    e_v, [jnp.full((16,), base + ch * CH + k, jnp.int32)])
                for q in range(F // 16):
                    sbuf[b][k, pl.ds(q * 16, 16)] = (
                        gbuf[b][k, pl.ds(q * 16, 16)] * asp)
            return 0

        lax.fori_loop(0, CH // unroll, scale, 0)

    def gwait(b):
        pltpu.make_async_copy(wh_hbm.at[pl.ds(0, CH)], gbuf[b],
                              gs[b]).wait()

    def swait(b):
        pltpu.make_async_copy(wh_hbm.at[pl.ds(0, CH)], sbuf[b],
                              ss[b]).wait()

    for p in range(nphase):
        base = p * nph * CH
        pltpu.sync_copy(row3_hbm.at[w * nphase + p], row_v)
        pltpu.sync_copy(col3_hbm.at[w * nphase + p], col_v)
        pltpu.async_copy(wh_hbm.at[col_v.at[0]], gbuf[0], gs[0])
        pltpu.async_copy(wh_hbm.at[col_v.at[1]], gbuf[1], gs[1])

        def pair(j, _):
            for b in range(2):
                ch = 2 * j + b
                gwait(b)

                @pl.when(j >= 1)
                def _():
                    swait(b)

                scale_chunk(b, ch, base)
                pltpu.async_copy(sbuf[b], acc.at[row_v.at[ch]], ss[b],
                                 add=True)

                @pl.when(ch + 2 <= nph - 1)
                def _():
                    pltpu.async_copy(wh_hbm.at[col_v.at[ch + 2]], gbuf[b],
                                     gs[b])
            return 0

        lax.fori_loop(0, (nph - 1) // 2, pair, 0)
        chl = nph - 1
        gwait(0)
        swait(0)
        scale_chunk(0, chl, base)
        pltpu.async_copy(sbuf[0], acc.at[row_v.at[chl]], ss[0], add=True)
        swait(0)
        swait(1)

    plsc.subcore_barrier()
    pltpu.sync_copy(acc.at[pl.ds(sid * rows_per_tile, rows_per_tile)],
                    pout_hbm.at[cid, pl.ds(sid * rows_per_tile,
                                           rows_per_tile)])


def kernel(h, edge_index, W, a):
    n, f = h.shape
    e_cnt = edge_index.shape[1]
    ept = e_cnt // NW
    nph = ept // (NPHASE * CH)

    row = edge_index[0].astype(jnp.int32)
    col = edge_index[1].astype(jnp.int32)
    row2 = row.reshape(NW, ept)
    col2 = col.reshape(NW, ept)
    row3 = row.reshape(NW * NPHASE, nph, CH)
    col3 = col.reshape(NW * NPHASE, nph, CH)

    wh, s = pl.pallas_call(
        _tc_mm_body,
        out_shape=(jax.ShapeDtypeStruct((n, f), jnp.float32),
                   jax.ShapeDtypeStruct((2, n), jnp.float32)),
    )(h, W, a)

    mesh = plsc.VectorSubcoreMesh(core_axis_name="c", subcore_axis_name="s",
                                  num_cores=NC, num_subcores=NS)
    sc_params = pltpu.CompilerParams(needs_layout_passes=False,
                                     use_tc_tiling_on_sc=False)

    scores = functools.partial(
        pl.kernel,
        mesh=mesh,
        out_type=(jax.ShapeDtypeStruct((NW, ept), jnp.float32),
                  jax.ShapeDtypeStruct((NW, 2, 16), jnp.float32)),
        scratch_types=[
            pltpu.VMEM((ept,), jnp.int32),
            pltpu.VMEM((ept,), jnp.int32),
            pltpu.VMEM((n,), jnp.float32),
            pltpu.VMEM((n,), jnp.float32),
            pltpu.VMEM((ept,), jnp.float32),
            pltpu.VMEM((2, 16), jnp.float32),
        ],
        compiler_params=sc_params,
    )(_sc_scores_body)
    e_all, parts = scores(row2, col2, s)

    aggr = functools.partial(
        pl.kernel,
        mesh=mesh,
        out_type=jax.ShapeDtypeStruct((NC, n, f), jnp.float32),
        scratch_types=[
            pltpu.VMEM_SHARED((n, f), jnp.float32),
            pltpu.VMEM((ept,), jnp.float32),
            pltpu.VMEM((nph, CH), jnp.int32),
            pltpu.VMEM((nph, CH), jnp.int32),
            pltpu.VMEM((CH, f), jnp.float32),
            pltpu.VMEM((CH, f), jnp.float32),
            pltpu.VMEM((CH, f), jnp.float32),
            pltpu.VMEM((CH, f), jnp.float32),
            pltpu.VMEM((NW, 2, 16), jnp.float32),
            pltpu.SemaphoreType.DMA,
            pltpu.SemaphoreType.DMA,
            pltpu.SemaphoreType.DMA,
            pltpu.SemaphoreType.DMA,
        ],
        compiler_params=sc_params,
    )(_sc_aggr_body)
    pout = aggr(wh, e_all, parts, row3, col3)

    out = pl.pallas_call(
        _tc_add_body,
        out_shape=jax.ShapeDtypeStruct((n, f), jnp.float32),
    )(pout)
    return out


# speedup vs baseline: 2.0145x; 2.0145x over previous
import functools

import jax
import jax.numpy as jnp
from jax import lax
from jax.experimental import pallas as pl
from jax.experimental.pallas import tpu as pltpu
from jax.experimental.pallas import tpu_sc as plsc

ALPHA = 0.2
NC = 2
NS = 16
NW = NC * NS
CH = 40
NPHASE = 2
F = 128


def _tc_mm_body(h_ref, w_ref, a_ref, wh_ref, s_ref):
    hh = h_ref[...]
    ww = w_ref[...]
    wh = lax.dot_general(hh, ww, (((1,), (1,)), ((), ())),
                         preferred_element_type=jnp.float32)
    wh_ref[...] = wh
    am = a_ref[...].reshape(2, F)
    s_ref[...] = lax.dot_general(am, wh, (((1,), (1,)), ((), ())),
                                 preferred_element_type=jnp.float32)


def _tc_add_body(p_ref, o_ref):
    o_ref[...] = p_ref[0] + p_ref[1]


def _sc_scores_body(row_hbm, col_hbm, s_hbm, e_hbm, part_hbm,
                    row_v, col_v, s1_v, s2_v, e_v, part_v):
    ept = row_v.shape[0]
    cid = lax.axis_index("c")
    sid = lax.axis_index("s")
    w = cid * NS + sid

    pltpu.sync_copy(row_hbm.at[w], row_v)
    pltpu.sync_copy(col_hbm.at[w], col_v)
    pltpu.sync_copy(s_hbm.at[0], s1_v)
    pltpu.sync_copy(s_hbm.at[1], s2_v)

    neg = jnp.full((16,), -1e30, jnp.float32)

    def pass1(i, m_acc):
        r = row_v[pl.ds(i * 16, 16)]
        c = col_v[pl.ds(i * 16, 16)]
        sr = plsc.load_gather(s1_v, [r])
        sc = plsc.load_gather(s2_v, [c])
        x = sr + sc
        e = jnp.where(x > 0, x, ALPHA * x)
        e_v[pl.ds(i * 16, 16)] = e
        return jnp.maximum(m_acc, e)

    m_vec = lax.fori_loop(0, ept // 16, pass1, neg)
    m16 = jnp.full((16,), jnp.max(m_vec))

    def pass2(i, s_acc):
        return s_acc + jnp.exp(e_v[pl.ds(i * 16, 16)] - m16)

    s_vec = lax.fori_loop(0, ept // 16, pass2, jnp.zeros((16,), jnp.float32))
    part_v[0, :] = m16
    part_v[1, :] = jnp.full((16,), jnp.sum(s_vec))

    pltpu.sync_copy(e_v, e_hbm.at[w])
    pltpu.sync_copy(part_v, part_hbm.at[w])


def _sc_aggr_body(wh_hbm, e_hbm, part_hbm, row3_hbm, col3_hbm, pout_hbm,
                  acc, e_v, row_v, col_v, g0, g1, s0, s1_, part_v,
                  gs0, gs1, ss0, ss1):
    n = acc.shape[0]
    ept = e_v.shape[0]
    nph = row3_hbm.shape[1]
    nphase = row3_hbm.shape[0] // NW
    rows_per_tile = n // NS
    cid = lax.axis_index("c")
    sid = lax.axis_index("s")
    w = cid * NS + sid
    gbuf = (g0, g1)
    sbuf = (s0, s1_)
    gs = (gs0, gs1)
    ss = (ss0, ss1)

    pltpu.sync_copy(e_hbm.at[w], e_v)
    pltpu.sync_copy(part_hbm, part_v)

    def mred(t, m_acc):
        return jnp.maximum(m_acc, part_v[t, 0, :])

    m_all = lax.fori_loop(0, NW, mred, jnp.full((16,), -1e30, jnp.float32))
    m16 = jnp.full((16,), jnp.max(m_all))

    def sred(t, s_acc):
        return s_acc + part_v[t, 1, :] * jnp.exp(part_v[t, 0, :] - m16)

    s_all = lax.fori_loop(0, NW, sred, jnp.zeros((16,), jnp.float32))
    inv_s = 1.0 / s_all

    def att_fill(i, _):
        v = e_v[pl.ds(i * 16, 16)]
        e_v[pl.ds(i * 16, 16)] = jnp.exp(v - m16) * inv_s
        return 0

    lax.fori_loop(0, ept // 16, att_fill, 0)

    def zfill(i, _):
        for q in range(F // 16):
            s0[i, pl.ds(q * 16, 16)] = jnp.zeros((16,), jnp.float32)
        return 0

    lax.fori_loop(0, CH, zfill, 0)

    def zcopy(k, _):
        pltpu.sync_copy(s0, acc.at[pl.ds(sid * rows_per_tile + k * CH, CH)])
        return 0

    lax.fori_loop(0, rows_per_tile // CH, zcopy, 0)
    rem = rows_per_tile % CH
    if rem:
        pltpu.sync_copy(
            s0.at[pl.ds(0, rem)],
            acc.at[pl.ds(sid * rows_per_tile + CH * (rows_per_tile // CH),
                         rem)])
    plsc.subcore_barrier()

    def scale_chunk(b, ch, base):
        @plsc.parallel_loop(0, CH, unroll=2)
        def scale(k):
            asp = plsc.load_gather(
                e_v, [jnp.full((16,), base + ch * CH + k, jnp.int32)])
            for q in range(F // 16):
                sbuf[b][k, pl.ds(q * 16, 16)] = (
                    gbuf[b][k, pl.ds(q * 16, 16)] * asp)

    def gwait(b):
        pltpu.make_async_copy(wh_hbm.at[pl.ds(0, CH)], gbuf[b],
                              gs[b]).wait()

    def swait(b):
        pltpu.make_async_copy(wh_hbm.at[pl.ds(0, CH)], sbuf[b],
                              ss[b]).wait()

    for p in range(nphase):
        base = p * nph * CH
        pltpu.sync_copy(row3_hbm.at[w * nphase + p], row_v)
        pltpu.sync_copy(col3_hbm.at[w * nphase + p], col_v)
        pltpu.async_copy(wh_hbm.at[col_v.at[0]], gbuf[0], gs[0])
        pltpu.async_copy(wh_hbm.at[col_v.at[1]], gbuf[1], gs[1])

        def pair(j, _):
            for b in range(2):
                ch = 2 * j + b
                gwait(b)

                @pl.when(j >= 1)
                def _():
                    swait(b)

                scale_chunk(b, ch, base)
                pltpu.async_copy(sbuf[b], acc.at[row_v.at[ch]], ss[b],
                                 add=True)

                @pl.when(ch + 2 <= nph - 1)
                def _():
                    pltpu.async_copy(wh_hbm.at[col_v.at[ch + 2]], gbuf[b],
                                     gs[b])
            return 0

        lax.fori_loop(0, (nph - 1) // 2, pair, 0)
        chl = nph - 1
        gwait(0)
        swait(0)
        scale_chunk(0, chl, base)
        pltpu.async_copy(sbuf[0], acc.at[row_v.at[chl]], ss[0], add=True)
        swait(0)
        swait(1)

    plsc.subcore_barrier()
    pltpu.sync_copy(acc.at[pl.ds(sid * rows_per_tile, rows_per_tile)],
                    pout_hbm.at[cid, pl.ds(sid * rows_per_tile,
                                           rows_per_tile)])


def kernel(h, edge_index, W, a):
    n, f = h.shape
    e_cnt = edge_index.shape[1]
    ept = e_cnt // NW
    nph = ept // (NPHASE * CH)

    row = edge_index[0].astype(jnp.int32)
    col = edge_index[1].astype(jnp.int32)
    row2 = row.reshape(NW, ept)
    col2 = col.reshape(NW, ept)
    row3 = row.reshape(NW * NPHASE, nph, CH)
    col3 = col.reshape(NW * NPHASE, nph, CH)

    wh, s = pl.pallas_call(
        _tc_mm_body,
        out_shape=(jax.ShapeDtypeStruct((n, f), jnp.float32),
                   jax.ShapeDtypeStruct((2, n), jnp.float32)),
    )(h, W, a)

    mesh = plsc.VectorSubcoreMesh(core_axis_name="c", subcore_axis_name="s",
                                  num_cores=NC, num_subcores=NS)
    sc_params = pltpu.CompilerParams(needs_layout_passes=False,
                                     use_tc_tiling_on_sc=False)

    scores = functools.partial(
        pl.kernel,
        mesh=mesh,
        out_type=(jax.ShapeDtypeStruct((NW, ept), jnp.float32),
                  jax.ShapeDtypeStruct((NW, 2, 16), jnp.float32)),
        scratch_types=[
            pltpu.VMEM((ept,), jnp.int32),
            pltpu.VMEM((ept,), jnp.int32),
            pltpu.VMEM((n,), jnp.float32),
            pltpu.VMEM((n,), jnp.float32),
            pltpu.VMEM((ept,), jnp.float32),
            pltpu.VMEM((2, 16), jnp.float32),
        ],
        compiler_params=sc_params,
    )(_sc_scores_body)
    e_all, parts = scores(row2, col2, s)

    aggr = functools.partial(
        pl.kernel,
        mesh=mesh,
        out_type=jax.ShapeDtypeStruct((NC, n, f), jnp.float32),
        scratch_types=[
            pltpu.VMEM_SHARED((n, f), jnp.float32),
            pltpu.VMEM((ept,), jnp.float32),
            pltpu.VMEM((nph, CH), jnp.int32),
            pltpu.VMEM((nph, CH), jnp.int32),
            pltpu.VMEM((CH, f), jnp.float32),
            pltpu.VMEM((CH, f), jnp.float32),
            pltpu.VMEM((CH, f), jnp.float32),
            pltpu.VMEM((CH, f), jnp.float32),
            pltpu.VMEM((NW, 2, 16), jnp.float32),
            pltpu.SemaphoreType.DMA,
            pltpu.SemaphoreType.DMA,
            pltpu.SemaphoreType.DMA,
            pltpu.SemaphoreType.DMA,
        ],
        compiler_params=sc_params,
    )(_sc_aggr_body)
    pout = aggr(wh, e_all, parts, row3, col3)

    out = pl.pallas_call(
        _tc_add_body,
        out_shape=jax.ShapeDtypeStruct((n, f), jnp.float32),
    )(pout)
    return out


# speedup vs baseline: 2.0245x; 1.0050x over previous
import functools

import jax
import jax.numpy as jnp
from jax import lax
from jax.experimental import pallas as pl
from jax.experimental.pallas import tpu as pltpu
from jax.experimental.pallas import tpu_sc as plsc

ALPHA = 0.2
NC = 2
NS = 16
NW = NC * NS
CH = 40
NPHASE = 2
F = 128


def _tc_mm_body(h_ref, w_ref, a_ref, wh_ref, s_ref):
    hh = h_ref[...]
    ww = w_ref[...]
    wh = lax.dot_general(hh, ww, (((1,), (1,)), ((), ())),
                         preferred_element_type=jnp.float32)
    wh_ref[...] = wh
    am = a_ref[...].reshape(2, F)
    s_ref[...] = lax.dot_general(am, wh, (((1,), (1,)), ((), ())),
                                 preferred_element_type=jnp.float32)


def _tc_add_body(p_ref, o_ref):
    o_ref[...] = p_ref[0] + p_ref[1]


def _sc_scores_body(row_hbm, col_hbm, s_hbm, e_hbm, part_hbm,
                    row_v, col_v, s1_v, s2_v, e_v, part_v):
    ept = row_v.shape[0]
    cid = lax.axis_index("c")
    sid = lax.axis_index("s")
    w = cid * NS + sid

    pltpu.sync_copy(row_hbm.at[w], row_v)
    pltpu.sync_copy(col_hbm.at[w], col_v)
    pltpu.sync_copy(s_hbm.at[0], s1_v)
    pltpu.sync_copy(s_hbm.at[1], s2_v)

    neg = jnp.full((16,), -1e30, jnp.float32)

    def pass1(i, m_acc):
        r = row_v[pl.ds(i * 16, 16)]
        c = col_v[pl.ds(i * 16, 16)]
        sr = plsc.load_gather(s1_v, [r])
        sc = plsc.load_gather(s2_v, [c])
        x = sr + sc
        e = jnp.where(x > 0, x, ALPHA * x)
        e_v[pl.ds(i * 16, 16)] = e
        return jnp.maximum(m_acc, e)

    m_vec = lax.fori_loop(0, ept // 16, pass1, neg)
    m16 = jnp.full((16,), jnp.max(m_vec))

    def pass2(i, s_acc):
        return s_acc + jnp.exp(e_v[pl.ds(i * 16, 16)] - m16)

    s_vec = lax.fori_loop(0, ept // 16, pass2, jnp.zeros((16,), jnp.float32))
    part_v[0, :] = m16
    part_v[1, :] = jnp.full((16,), jnp.sum(s_vec))

    pltpu.sync_copy(e_v, e_hbm.at[w])
    pltpu.sync_copy(part_v, part_hbm.at[w])


def _sc_aggr_body(wh_hbm, e_hbm, part_hbm, row3_hbm, col3_hbm, pout_hbm,
                  acc, e_v, row_v, col_v, g0, g1, s0, s1_, part_v,
                  gs0, gs1, ss0, ss1):
    n = acc.shape[0]
    ept = e_v.shape[0]
    nph = row3_hbm.shape[1]
    nphase = row3_hbm.shape[0] // NW
    rows_per_tile = n // NS
    cid = lax.axis_index("c")
    sid = lax.axis_index("s")
    w = cid * NS + sid
    gbuf = (g0, g1)
    sbuf = (s0, s1_)
    gs = (gs0, gs1)
    ss = (ss0, ss1)

    pltpu.sync_copy(e_hbm.at[w], e_v)
    pltpu.sync_copy(part_hbm, part_v)

    def mred(t, m_acc):
        return jnp.maximum(m_acc, part_v[t, 0, :])

    m_all = lax.fori_loop(0, NW, mred, jnp.full((16,), -1e30, jnp.float32))
    m16 = jnp.full((16,), jnp.max(m_all))

    def sred(t, s_acc):
        return s_acc + part_v[t, 1, :] * jnp.exp(part_v[t, 0, :] - m16)

    s_all = lax.fori_loop(0, NW, sred, jnp.zeros((16,), jnp.float32))
    inv_s = 1.0 / s_all

    def att_fill(i, _):
        v = e_v[pl.ds(i * 16, 16)]
        e_v[pl.ds(i * 16, 16)] = jnp.exp(v - m16) * inv_s
        return 0

    lax.fori_loop(0, ept // 16, att_fill, 0)

    def zfill(i, _):
        for q in range(F // 16):
            s0[i, pl.ds(q * 16, 16)] = jnp.zeros((16,), jnp.float32)
        return 0

    lax.fori_loop(0, CH, zfill, 0)

    def zcopy(k, _):
        pltpu.sync_copy(s0, acc.at[pl.ds(sid * rows_per_tile + k * CH, CH)])
        return 0

    lax.fori_loop(0, rows_per_tile // CH, zcopy, 0)
    rem = rows_per_tile % CH
    if rem:
        pltpu.sync_copy(
            s0.at[pl.ds(0, rem)],
            acc.at[pl.ds(sid * rows_per_tile + CH * (rows_per_tile // CH),
                         rem)])
    plsc.subcore_barrier()

    def scale_chunk(b, ch, base):
        @plsc.parallel_loop(0, CH, unroll=4)
        def scale(k):
            asp = plsc.load_gather(
                e_v, [jnp.full((16,), base + ch * CH + k, jnp.int32)])
            for q in range(F // 16):
                sbuf[b][k, pl.ds(q * 16, 16)] = (
                    gbuf[b][k, pl.ds(q * 16, 16)] * asp)

    def gwait(b):
        pltpu.make_async_copy(wh_hbm.at[pl.ds(0, CH)], gbuf[b],
                              gs[b]).wait()

    def swait(b):
        pltpu.make_async_copy(wh_hbm.at[pl.ds(0, CH)], sbuf[b],
                              ss[b]).wait()

    for p in range(nphase):
        base = p * nph * CH
        pltpu.sync_copy(row3_hbm.at[w * nphase + p], row_v)
        pltpu.sync_copy(col3_hbm.at[w * nphase + p], col_v)
        pltpu.async_copy(wh_hbm.at[col_v.at[0]], gbuf[0], gs[0])
        pltpu.async_copy(wh_hbm.at[col_v.at[1]], gbuf[1], gs[1])

        def pair(j, _):
            for b in range(2):
                ch = 2 * j + b
                gwait(b)

                @pl.when(j >= 1)
                def _():
                    swait(b)

                scale_chunk(b, ch, base)
                pltpu.async_copy(sbuf[b], acc.at[row_v.at[ch]], ss[b],
                                 add=True)

                @pl.when(ch + 2 <= nph - 1)
                def _():
                    pltpu.async_copy(wh_hbm.at[col_v.at[ch + 2]], gbuf[b],
                                     gs[b])
            return 0

        lax.fori_loop(0, (nph - 1) // 2, pair, 0)
        chl = nph - 1
        gwait(0)
        swait(0)
        scale_chunk(0, chl, base)
        pltpu.async_copy(sbuf[0], acc.at[row_v.at[chl]], ss[0], add=True)
        swait(0)
        swait(1)

    plsc.subcore_barrier()
    pltpu.sync_copy(acc.at[pl.ds(sid * rows_per_tile, rows_per_tile)],
                    pout_hbm.at[cid, pl.ds(sid * rows_per_tile,
                                           rows_per_tile)])


def kernel(h, edge_index, W, a):
    n, f = h.shape
    e_cnt = edge_index.shape[1]
    ept = e_cnt // NW
    nph = ept // (NPHASE * CH)

    row = edge_index[0].astype(jnp.int32)
    col = edge_index[1].astype(jnp.int32)
    row2 = row.reshape(NW, ept)
    col2 = col.reshape(NW, ept)
    row3 = row.reshape(NW * NPHASE, nph, CH)
    col3 = col.reshape(NW * NPHASE, nph, CH)

    wh, s = pl.pallas_call(
        _tc_mm_body,
        out_shape=(jax.ShapeDtypeStruct((n, f), jnp.float32),
                   jax.ShapeDtypeStruct((2, n), jnp.float32)),
    )(h, W, a)

    mesh = plsc.VectorSubcoreMesh(core_axis_name="c", subcore_axis_name="s",
                                  num_cores=NC, num_subcores=NS)
    sc_params = pltpu.CompilerParams(needs_layout_passes=False,
                                     use_tc_tiling_on_sc=False)

    scores = functools.partial(
        pl.kernel,
        mesh=mesh,
        out_type=(jax.ShapeDtypeStruct((NW, ept), jnp.float32),
                  jax.ShapeDtypeStruct((NW, 2, 16), jnp.float32)),
        scratch_types=[
            pltpu.VMEM((ept,), jnp.int32),
            pltpu.VMEM((ept,), jnp.int32),
            pltpu.VMEM((n,), jnp.float32),
            pltpu.VMEM((n,), jnp.float32),
            pltpu.VMEM((ept,), jnp.float32),
            pltpu.VMEM((2, 16), jnp.float32),
        ],
        compiler_params=sc_params,
    )(_sc_scores_body)
    e_all, parts = scores(row2, col2, s)

    aggr = functools.partial(
        pl.kernel,
        mesh=mesh,
        out_type=jax.ShapeDtypeStruct((NC, n, f), jnp.float32),
        scratch_types=[
            pltpu.VMEM_SHARED((n, f), jnp.float32),
            pltpu.VMEM((ept,), jnp.float32),
            pltpu.VMEM((nph, CH), jnp.int32),
            pltpu.VMEM((nph, CH), jnp.int32),
            pltpu.VMEM((CH, f), jnp.float32),
            pltpu.VMEM((CH, f), jnp.float32),
            pltpu.VMEM((CH, f), jnp.float32),
            pltpu.VMEM((CH, f), jnp.float32),
            pltpu.VMEM((NW, 2, 16), jnp.float32),
            pltpu.SemaphoreType.DMA,
            pltpu.SemaphoreType.DMA,
            pltpu.SemaphoreType.DMA,
            pltpu.SemaphoreType.DMA,
        ],
        compiler_params=sc_params,
    )(_sc_aggr_body)
    pout = aggr(wh, e_all, parts, row3, col3)

    out = pl.pallas_call(
        _tc_add_body,
        out_shape=jax.ShapeDtypeStruct((n, f), jnp.float32),
    )(pout)
    return out


# speedup vs baseline: 2.5504x; 1.2598x over previous
import functools

import jax
import jax.numpy as jnp
from jax import lax
from jax.experimental import pallas as pl
from jax.experimental.pallas import tpu as pltpu
from jax.experimental.pallas import tpu_sc as plsc

ALPHA = 0.2
NC = 2
NS = 16
NW = NC * NS
CH = 80
NPHASE = 5
F = 128


def _tc_mm_body(h_ref, w_ref, a_ref, whb_ref, s_ref):
    hh = h_ref[...]
    ww = w_ref[...]
    wh = lax.dot_general(hh, ww, (((1,), (1,)), ((), ())),
                         preferred_element_type=jnp.float32)
    whb_ref[...] = wh.astype(jnp.bfloat16)
    am = a_ref[...].reshape(2, F)
    s_ref[...] = lax.dot_general(am, wh, (((1,), (1,)), ((), ())),
                                 preferred_element_type=jnp.float32)


def _tc_add_body(p_ref, o_ref):
    o_ref[...] = p_ref[0] + p_ref[1]


def _sc_scores_body(row_hbm, col_hbm, s_hbm, e_hbm, part_hbm,
                    row_v, col_v, s1_v, s2_v, e_v, part_v):
    ept = row_v.shape[0]
    cid = lax.axis_index("c")
    sid = lax.axis_index("s")
    w = cid * NS + sid

    pltpu.sync_copy(row_hbm.at[w], row_v)
    pltpu.sync_copy(col_hbm.at[w], col_v)
    pltpu.sync_copy(s_hbm.at[0], s1_v)
    pltpu.sync_copy(s_hbm.at[1], s2_v)

    neg = jnp.full((16,), -1e30, jnp.float32)

    def pass1(i, m_acc):
        r = row_v[pl.ds(i * 16, 16)]
        c = col_v[pl.ds(i * 16, 16)]
        sr = plsc.load_gather(s1_v, [r])
        sc = plsc.load_gather(s2_v, [c])
        x = sr + sc
        e = jnp.where(x > 0, x, ALPHA * x)
        e_v[pl.ds(i * 16, 16)] = e
        return jnp.maximum(m_acc, e)

    m_vec = lax.fori_loop(0, ept // 16, pass1, neg)
    m16 = jnp.full((16,), jnp.max(m_vec))

    def pass2(i, s_acc):
        return s_acc + jnp.exp(e_v[pl.ds(i * 16, 16)] - m16)

    s_vec = lax.fori_loop(0, ept // 16, pass2, jnp.zeros((16,), jnp.float32))
    part_v[0, :] = m16
    part_v[1, :] = jnp.full((16,), jnp.sum(s_vec))

    pltpu.sync_copy(e_v, e_hbm.at[w])
    pltpu.sync_copy(part_v, part_hbm.at[w])


def _sc_aggr_body(wh_hbm, e_hbm, part_hbm, row3_hbm, col3_hbm, pout_hbm,
                  acc, e_v, row_v, col_v, g0, g1, s0, s1_, part_v,
                  gs0, gs1, ss0, ss1):
    n = acc.shape[0]
    ept = e_v.shape[0]
    nph = row3_hbm.shape[1]
    nphase = row3_hbm.shape[0] // NW
    rows_per_tile = n // NS
    cid = lax.axis_index("c")
    sid = lax.axis_index("s")
    w = cid * NS + sid
    gbuf = (g0, g1)
    sbuf = (s0, s1_)
    gs = (gs0, gs1)
    ss = (ss0, ss1)

    pltpu.sync_copy(e_hbm.at[w], e_v)
    pltpu.sync_copy(part_hbm, part_v)

    def mred(t, m_acc):
        return jnp.maximum(m_acc, part_v[t, 0, :])

    m_all = lax.fori_loop(0, NW, mred, jnp.full((16,), -1e30, jnp.float32))
    m16 = jnp.full((16,), jnp.max(m_all))

    def sred(t, s_acc):
        return s_acc + part_v[t, 1, :] * jnp.exp(part_v[t, 0, :] - m16)

    s_all = lax.fori_loop(0, NW, sred, jnp.zeros((16,), jnp.float32))
    inv_s = 1.0 / s_all

    def att_fill(i, _):
        v = e_v[pl.ds(i * 16, 16)]
        e_v[pl.ds(i * 16, 16)] = jnp.exp(v - m16) * inv_s
        return 0

    lax.fori_loop(0, ept // 16, att_fill, 0)

    def zfill(i, _):
        for q in range(F // 16):
            s0[i, pl.ds(q * 16, 16)] = jnp.zeros((16,), jnp.float32)
        return 0

    lax.fori_loop(0, CH, zfill, 0)

    def zcopy(k, _):
        pltpu.sync_copy(s0, acc.at[pl.ds(sid * rows_per_tile + k * CH, CH)])
        return 0

    lax.fori_loop(0, rows_per_tile // CH, zcopy, 0)
    rem = rows_per_tile % CH
    if rem:
        pltpu.sync_copy(
            s0.at[pl.ds(0, rem)],
            acc.at[pl.ds(sid * rows_per_tile + CH * (rows_per_tile // CH),
                         rem)])
    plsc.subcore_barrier()

    two_iota = 2 * lax.iota(jnp.int32, 16)

    def scale_chunk(b, ch, base):
        @plsc.parallel_loop(0, CH, unroll=2)
        def scale(k):
            asp = plsc.load_gather(
                e_v, [jnp.full((16,), base + ch * CH + k, jnp.int32)])
            kidx = jnp.full((16,), k, jnp.int32)
            for q in range(F // 32):
                wv = plsc.bitcast(gbuf[b][k, pl.ds(q * 32, 32)], jnp.int32)
                ev = plsc.bitcast(wv << 16, jnp.float32) * asp
                od = plsc.bitcast(wv & jnp.int32(-65536), jnp.float32) * asp
                plsc.store_scatter(sbuf[b], [kidx, two_iota + (32 * q)], ev)
                plsc.store_scatter(sbuf[b], [kidx, two_iota + (32 * q + 1)],
                                   od)

    def gwait(b):
        pltpu.make_async_copy(wh_hbm.at[pl.ds(0, CH)], gbuf[b],
                              gs[b]).wait()

    def swait(b):
        pltpu.make_async_copy(pout_hbm.at[0, pl.ds(0, CH)], sbuf[b],
                              ss[b]).wait()

    for p in range(nphase):
        base = p * nph * CH
        pltpu.sync_copy(row3_hbm.at[w * nphase + p], row_v)
        pltpu.sync_copy(col3_hbm.at[w * nphase + p], col_v)
        pltpu.async_copy(wh_hbm.at[col_v.at[0]], gbuf[0], gs[0])
        pltpu.async_copy(wh_hbm.at[col_v.at[1]], gbuf[1], gs[1])

        def pair(j, _):
            for b in range(2):
                ch = 2 * j + b
                gwait(b)

                @pl.when(j >= 1)
                def _():
                    swait(b)

                scale_chunk(b, ch, base)
                pltpu.async_copy(sbuf[b], acc.at[row_v.at[ch]], ss[b],
                                 add=True)

                @pl.when(ch + 2 <= nph - 1)
                def _():
                    pltpu.async_copy(wh_hbm.at[col_v.at[ch + 2]], gbuf[b],
                                     gs[b])
            return 0

        lax.fori_loop(0, (nph - 1) // 2, pair, 0)
        chl = nph - 1
        gwait(0)
        swait(0)
        scale_chunk(0, chl, base)
        pltpu.async_copy(sbuf[0], acc.at[row_v.at[chl]], ss[0], add=True)
        swait(0)
        swait(1)

    plsc.subcore_barrier()
    pltpu.sync_copy(acc.at[pl.ds(sid * rows_per_tile, rows_per_tile)],
                    pout_hbm.at[cid, pl.ds(sid * rows_per_tile,
                                           rows_per_tile)])


def kernel(h, edge_index, W, a):
    n, f = h.shape
    e_cnt = edge_index.shape[1]
    ept = e_cnt // NW
    nph = ept // (NPHASE * CH)

    row = edge_index[0].astype(jnp.int32)
    col = edge_index[1].astype(jnp.int32)
    row2 = row.reshape(NW, ept)
    col2 = col.reshape(NW, ept)
    row3 = row.reshape(NW * NPHASE, nph, CH)
    col3 = col.reshape(NW * NPHASE, nph, CH)

    whb, s = pl.pallas_call(
        _tc_mm_body,
        out_shape=(jax.ShapeDtypeStruct((n, f), jnp.bfloat16),
                   jax.ShapeDtypeStruct((2, n), jnp.float32)),
    )(h, W, a)

    mesh = plsc.VectorSubcoreMesh(core_axis_name="c", subcore_axis_name="s",
                                  num_cores=NC, num_subcores=NS)
    sc_params = pltpu.CompilerParams(needs_layout_passes=False,
                                     use_tc_tiling_on_sc=False)

    scores = functools.partial(
        pl.kernel,
        mesh=mesh,
        out_type=(jax.ShapeDtypeStruct((NW, ept), jnp.float32),
                  jax.ShapeDtypeStruct((NW, 2, 16), jnp.float32)),
        scratch_types=[
            pltpu.VMEM((ept,), jnp.int32),
            pltpu.VMEM((ept,), jnp.int32),
            pltpu.VMEM((n,), jnp.float32),
            pltpu.VMEM((n,), jnp.float32),
            pltpu.VMEM((ept,), jnp.float32),
            pltpu.VMEM((2, 16), jnp.float32),
        ],
        compiler_params=sc_params,
    )(_sc_scores_body)
    e_all, parts = scores(row2, col2, s)

    aggr = functools.partial(
        pl.kernel,
        mesh=mesh,
        out_type=jax.ShapeDtypeStruct((NC, n, f), jnp.float32),
        scratch_types=[
            pltpu.VMEM_SHARED((n, f), jnp.float32),
            pltpu.VMEM((ept,), jnp.float32),
            pltpu.VMEM((nph, CH), jnp.int32),
            pltpu.VMEM((nph, CH), jnp.int32),
            pltpu.VMEM((CH, f), jnp.bfloat16),
            pltpu.VMEM((CH, f), jnp.bfloat16),
            pltpu.VMEM((CH, f), jnp.float32),
            pltpu.VMEM((CH, f), jnp.float32),
            pltpu.VMEM((NW, 2, 16), jnp.float32),
            pltpu.SemaphoreType.DMA,
            pltpu.SemaphoreType.DMA,
            pltpu.SemaphoreType.DMA,
            pltpu.SemaphoreType.DMA,
        ],
        compiler_params=sc_params,
    )(_sc_aggr_body)
    pout = aggr(whb, e_all, parts, row3, col3)

    out = pl.pallas_call(
        _tc_add_body,
        out_shape=jax.ShapeDtypeStruct((n, f), jnp.float32),
    )(pout)
    return out


# speedup vs baseline: 2.6411x; 1.0356x over previous
import functools

import jax
import jax.numpy as jnp
from jax import lax
from jax.experimental import pallas as pl
from jax.experimental.pallas import tpu as pltpu
from jax.experimental.pallas import tpu_sc as plsc

ALPHA = 0.2
NC = 2
NS = 16
NW = NC * NS
CH = 80
NPHASE = 5
F = 128


def _tc_s_body(h_ref, w_ref, a_ref, s_ref):
    am = a_ref[...].reshape(2, F)
    v = lax.dot_general(am, w_ref[...], (((1,), (0,)), ((), ())),
                        preferred_element_type=jnp.float32)
    s_ref[...] = lax.dot_general(v, h_ref[...], (((1,), (1,)), ((), ())),
                                 preferred_element_type=jnp.float32)


def _tc_whb_body(h_ref, w_ref, whb_ref):
    wh = lax.dot_general(h_ref[...], w_ref[...], (((1,), (1,)), ((), ())),
                         preferred_element_type=jnp.float32)
    whb_ref[...] = wh.astype(jnp.bfloat16)


def _tc_add_body(p_ref, o_ref):
    o_ref[...] = p_ref[0] + p_ref[1]


def _sc_scores_body(row_hbm, col_hbm, s_hbm, e_hbm, part_hbm,
                    row_v, col_v, s1_v, s2_v, e_v, part_v):
    ept = row_v.shape[0]
    cid = lax.axis_index("c")
    sid = lax.axis_index("s")
    w = cid * NS + sid

    pltpu.sync_copy(row_hbm.at[w], row_v)
    pltpu.sync_copy(col_hbm.at[w], col_v)
    pltpu.sync_copy(s_hbm.at[0], s1_v)
    pltpu.sync_copy(s_hbm.at[1], s2_v)

    neg = jnp.full((16,), -1e30, jnp.float32)

    def pass1(i, m_acc):
        r = row_v[pl.ds(i * 16, 16)]
        c = col_v[pl.ds(i * 16, 16)]
        sr = plsc.load_gather(s1_v, [r])
        sc = plsc.load_gather(s2_v, [c])
        x = sr + sc
        e = jnp.where(x > 0, x, ALPHA * x)
        e_v[pl.ds(i * 16, 16)] = e
        return jnp.maximum(m_acc, e)

    m_vec = lax.fori_loop(0, ept // 16, pass1, neg)
    m16 = jnp.full((16,), jnp.max(m_vec))

    def pass2(i, s_acc):
        return s_acc + jnp.exp(e_v[pl.ds(i * 16, 16)] - m16)

    s_vec = lax.fori_loop(0, ept // 16, pass2, jnp.zeros((16,), jnp.float32))
    part_v[0, :] = m16
    part_v[1, :] = jnp.full((16,), jnp.sum(s_vec))

    pltpu.sync_copy(e_v, e_hbm.at[w])
    pltpu.sync_copy(part_v, part_hbm.at[w])


def _sc_aggr_body(wh_hbm, e_hbm, part_hbm, row3_hbm, col3_hbm, pout_hbm,
                  acc, e_v, row_v, col_v, g0, g1, s0, s1_, part_v,
                  gs0, gs1, ss0, ss1):
    n = acc.shape[0]
    ept = e_v.shape[0]
    nph = row3_hbm.shape[1]
    nphase = row3_hbm.shape[0] // NW
    rows_per_tile = n // NS
    cid = lax.axis_index("c")
    sid = lax.axis_index("s")
    w = cid * NS + sid
    gbuf = (g0, g1)
    sbuf = (s0, s1_)
    gs = (gs0, gs1)
    ss = (ss0, ss1)

    def zfill(i, _):
        for q in range(F // 16):
            s0[i, pl.ds(q * 16, 16)] = jnp.zeros((16,), jnp.float32)
        return 0

    lax.fori_loop(0, CH, zfill, 0)
    nz = rows_per_tile // CH
    rem = rows_per_tile % CH
    for k in range(nz):
        pltpu.async_copy(s0, acc.at[pl.ds(sid * rows_per_tile + k * CH, CH)],
                         ss0)
    if rem:
        pltpu.async_copy(
            s0.at[pl.ds(0, rem)],
            acc.at[pl.ds(sid * rows_per_tile + CH * nz, rem)], ss0)

    pltpu.sync_copy(e_hbm.at[w], e_v)
    pltpu.sync_copy(part_hbm, part_v)

    def mred(t, m_acc):
        return jnp.maximum(m_acc, part_v[t, 0, :])

    m_all = lax.fori_loop(0, NW, mred, jnp.full((16,), -1e30, jnp.float32))
    m16 = jnp.full((16,), jnp.max(m_all))

    def sred(t, s_acc):
        return s_acc + part_v[t, 1, :] * jnp.exp(part_v[t, 0, :] - m16)

    s_all = lax.fori_loop(0, NW, sred, jnp.zeros((16,), jnp.float32))
    inv_s = 1.0 / s_all

    def att_fill(i, _):
        v = e_v[pl.ds(i * 16, 16)]
        e_v[pl.ds(i * 16, 16)] = jnp.exp(v - m16) * inv_s
        return 0

    lax.fori_loop(0, ept // 16, att_fill, 0)

    for k in range(nz):
        pltpu.make_async_copy(pout_hbm.at[0, pl.ds(0, CH)], s0, ss0).wait()
    if rem:
        pltpu.make_async_copy(pout_hbm.at[0, pl.ds(0, rem)],
                              s0.at[pl.ds(0, rem)], ss0).wait()
    plsc.subcore_barrier()

    two_iota = 2 * lax.iota(jnp.int32, 16)

    def scale_chunk(b, ch, base):
        @plsc.parallel_loop(0, CH, unroll=2)
        def scale(k):
            asp = plsc.load_gather(
                e_v, [jnp.full((16,), base + ch * CH + k, jnp.int32)])
            kidx = jnp.full((16,), k, jnp.int32)
            for q in range(F // 32):
                wv = plsc.bitcast(gbuf[b][k, pl.ds(q * 32, 32)], jnp.int32)
                ev = plsc.bitcast(wv << 16, jnp.float32) * asp
                od = plsc.bitcast(wv & jnp.int32(-65536), jnp.float32) * asp
                plsc.store_scatter(sbuf[b], [kidx, two_iota + (32 * q)], ev)
                plsc.store_scatter(sbuf[b], [kidx, two_iota + (32 * q + 1)],
                                   od)

    def gwait(b):
        pltpu.make_async_copy(wh_hbm.at[pl.ds(0, CH)], gbuf[b],
                              gs[b]).wait()

    def swait(b):
        pltpu.make_async_copy(pout_hbm.at[0, pl.ds(0, CH)], sbuf[b],
                              ss[b]).wait()

    for p in range(nphase):
        base = p * nph * CH
        pltpu.sync_copy(row3_hbm.at[w * nphase + p], row_v)
        pltpu.sync_copy(col3_hbm.at[w * nphase + p], col_v)
        pltpu.async_copy(wh_hbm.at[col_v.at[0]], gbuf[0], gs[0])
        pltpu.async_copy(wh_hbm.at[col_v.at[1]], gbuf[1], gs[1])

        def pair(j, _):
            for b in range(2):
                ch = 2 * j + b
                gwait(b)

                @pl.when(j >= 1)
                def _():
                    swait(b)

                scale_chunk(b, ch, base)
                pltpu.async_copy(sbuf[b], acc.at[row_v.at[ch]], ss[b],
                                 add=True)

                @pl.when(ch + 2 <= nph - 1)
                def _():
                    pltpu.async_copy(wh_hbm.at[col_v.at[ch + 2]], gbuf[b],
                                     gs[b])
            return 0

        lax.fori_loop(0, (nph - 1) // 2, pair, 0)
        chl = nph - 1
        gwait(0)
        swait(0)
        scale_chunk(0, chl, base)
        pltpu.async_copy(sbuf[0], acc.at[row_v.at[chl]], ss[0], add=True)
        swait(0)
        swait(1)

    plsc.subcore_barrier()
    pltpu.sync_copy(acc.at[pl.ds(sid * rows_per_tile, rows_per_tile)],
                    pout_hbm.at[cid, pl.ds(sid * rows_per_tile,
                                           rows_per_tile)])


def kernel(h, edge_index, W, a):
    n, f = h.shape
    e_cnt = edge_index.shape[1]
    ept = e_cnt // NW
    nph = ept // (NPHASE * CH)

    row = edge_index[0].astype(jnp.int32)
    col = edge_index[1].astype(jnp.int32)
    row2 = row.reshape(NW, ept)
    col2 = col.reshape(NW, ept)
    row3 = row.reshape(NW * NPHASE, nph, CH)
    col3 = col.reshape(NW * NPHASE, nph, CH)

    s = pl.pallas_call(
        _tc_s_body,
        out_shape=jax.ShapeDtypeStruct((2, n), jnp.float32),
    )(h, W, a)
    whb = pl.pallas_call(
        _tc_whb_body,
        out_shape=jax.ShapeDtypeStruct((n, f), jnp.bfloat16),
    )(h, W)

    mesh = plsc.VectorSubcoreMesh(core_axis_name="c", subcore_axis_name="s",
                                  num_cores=NC, num_subcores=NS)
    sc_params = pltpu.CompilerParams(needs_layout_passes=False,
                                     use_tc_tiling_on_sc=False)

    scores = functools.partial(
        pl.kernel,
        mesh=mesh,
        out_type=(jax.ShapeDtypeStruct((NW, ept), jnp.float32),
                  jax.ShapeDtypeStruct((NW, 2, 16), jnp.float32)),
        scratch_types=[
            pltpu.VMEM((ept,), jnp.int32),
            pltpu.VMEM((ept,), jnp.int32),
            pltpu.VMEM((n,), jnp.float32),
            pltpu.VMEM((n,), jnp.float32),
            pltpu.VMEM((ept,), jnp.float32),
            pltpu.VMEM((2, 16), jnp.float32),
        ],
        compiler_params=sc_params,
    )(_sc_scores_body)
    e_all, parts = scores(row2, col2, s)

    aggr = functools.partial(
        pl.kernel,
        mesh=mesh,
        out_type=jax.ShapeDtypeStruct((NC, n, f), jnp.float32),
        scratch_types=[
            pltpu.VMEM_SHARED((n, f), jnp.float32),
            pltpu.VMEM((ept,), jnp.float32),
            pltpu.VMEM((nph, CH), jnp.int32),
            pltpu.VMEM((nph, CH), jnp.int32),
            pltpu.VMEM((CH, f), jnp.bfloat16),
            pltpu.VMEM((CH, f), jnp.bfloat16),
            pltpu.VMEM((CH, f), jnp.float32),
            pltpu.VMEM((CH, f), jnp.float32),
            pltpu.VMEM((NW, 2, 16), jnp.float32),
            pltpu.SemaphoreType.DMA,
            pltpu.SemaphoreType.DMA,
            pltpu.SemaphoreType.DMA,
            pltpu.SemaphoreType.DMA,
        ],
        compiler_params=sc_params,
    )(_sc_aggr_body)
    pout = aggr(whb, e_all, parts, row3, col3)

    out = pl.pallas_call(
        _tc_add_body,
        out_shape=jax.ShapeDtypeStruct((n, f), jnp.float32),
    )(pout)
    return out
